# baseline (device time: 103631 ns/iter reference)
import jax
import jax.numpy as jnp
from jax import lax
from jax.experimental import pallas as pl
from jax.experimental.pallas import tpu as pltpu

N_DEV = 8


def kernel(x, w_mat):
    m, k = x.shape
    _, n = w_mat.shape
    chunk = m // N_DEV

    def body(x_ref, w_ref, out_ref, recv_buf, send_buf, send_sems, recv_sems):
        my = lax.axis_index("i")
        left = lax.rem(my + N_DEV - 1, N_DEV)
        right = lax.rem(my + 1, N_DEV)

        barrier_sem = pltpu.get_barrier_semaphore()
        for nbr in (left, right):
            pl.semaphore_signal(
                barrier_sem, inc=1,
                device_id=(nbr,), device_id_type=pl.DeviceIdType.MESH,
            )
        pl.semaphore_wait(barrier_sem, 2)

        def chunk_partial(c):
            xs = x_ref[pl.ds(c * chunk, chunk), :]
            return lax.dot_general(
                xs, w_ref[:, :],
                (((1,), (0,)), ((), ())),
                preferred_element_type=jnp.float32,
            )

        c0 = lax.rem(my + N_DEV - 1, N_DEV)
        send_buf[0, :, :] = chunk_partial(c0).astype(jnp.bfloat16)

        acc = None
        for s in range(N_DEV - 1):
            rdma = pltpu.make_async_remote_copy(
                src_ref=send_buf.at[s],
                dst_ref=recv_buf.at[s],
                send_sem=send_sems.at[s],
                recv_sem=recv_sems.at[s],
                device_id=(right,),
                device_id_type=pl.DeviceIdType.MESH,
            )
            rdma.start()
            rdma.wait()
            c = lax.rem(my + 2 * N_DEV - 2 - s, N_DEV)
            acc = chunk_partial(c) + recv_buf[s, :, :].astype(jnp.float32)
            if s < N_DEV - 2:
                send_buf[s + 1, :, :] = acc.astype(jnp.bfloat16)

        out_ref[:, :] = acc / (1.0 + jnp.exp(-jnp.clip(acc, -60.0, 60.0)))

    return pl.pallas_call(
        body,
        out_shape=jax.ShapeDtypeStruct((chunk, n), jnp.float32),
        in_specs=[
            pl.BlockSpec(memory_space=pltpu.VMEM),
            pl.BlockSpec(memory_space=pltpu.VMEM),
        ],
        out_specs=pl.BlockSpec(memory_space=pltpu.VMEM),
        scratch_shapes=[
            pltpu.VMEM((N_DEV - 1, chunk, n), jnp.bfloat16),
            pltpu.VMEM((N_DEV - 1, chunk, n), jnp.bfloat16),
            pltpu.SemaphoreType.DMA((N_DEV - 1,)),
            pltpu.SemaphoreType.DMA((N_DEV - 1,)),
        ],
        compiler_params=pltpu.CompilerParams(collective_id=0),
    )(x, w_mat)


# device time: 67534 ns/iter; 1.5345x vs baseline; 1.5345x over previous
import jax
import jax.numpy as jnp
from jax import lax
from jax.experimental import pallas as pl
from jax.experimental.pallas import tpu as pltpu

N_DEV = 8


def kernel(x, w_mat):
    m, k = x.shape
    _, n = w_mat.shape
    chunk = m // N_DEV
    half = n // 2

    def body(x_ref, w_ref, out_ref,
             recv_r, recv_l, send_r, send_l,
             send_sems_r, recv_sems_r, send_sems_l, recv_sems_l):
        my = lax.axis_index("i")
        left = lax.rem(my + N_DEV - 1, N_DEV)
        right = lax.rem(my + 1, N_DEV)

        barrier_sem = pltpu.get_barrier_semaphore()
        for nbr in (left, right):
            pl.semaphore_signal(
                barrier_sem, inc=1,
                device_id=(nbr,), device_id_type=pl.DeviceIdType.MESH,
            )
        pl.semaphore_wait(barrier_sem, 2)

        def partial_r(c):
            xs = x_ref[pl.ds(c * chunk, chunk), :]
            return lax.dot_general(
                xs, w_ref[:, :half],
                (((1,), (0,)), ((), ())),
                preferred_element_type=jnp.float32,
            )

        def partial_l(c):
            xs = x_ref[pl.ds(c * chunk, chunk), :]
            return lax.dot_general(
                xs, w_ref[:, half:],
                (((1,), (0,)), ((), ())),
                preferred_element_type=jnp.float32,
            )

        send_r[0, :, :] = partial_r(lax.rem(my + N_DEV - 1, N_DEV)).astype(jnp.bfloat16)
        send_l[0, :, :] = partial_l(lax.rem(my + 1, N_DEV)).astype(jnp.bfloat16)

        acc_r = acc_l = None
        for s in range(N_DEV - 1):
            rdma_r = pltpu.make_async_remote_copy(
                src_ref=send_r.at[s],
                dst_ref=recv_r.at[s],
                send_sem=send_sems_r.at[s],
                recv_sem=recv_sems_r.at[s],
                device_id=(right,),
                device_id_type=pl.DeviceIdType.MESH,
            )
            rdma_l = pltpu.make_async_remote_copy(
                src_ref=send_l.at[s],
                dst_ref=recv_l.at[s],
                send_sem=send_sems_l.at[s],
                recv_sem=recv_sems_l.at[s],
                device_id=(left,),
                device_id_type=pl.DeviceIdType.MESH,
            )
            rdma_r.start()
            rdma_l.start()
            pr = partial_r(lax.rem(my + 2 * N_DEV - 2 - s, N_DEV))
            pll = partial_l(lax.rem(my + 2 + s, N_DEV))
            rdma_r.wait()
            rdma_l.wait()
            acc_r = pr + recv_r[s, :, :].astype(jnp.float32)
            acc_l = pll + recv_l[s, :, :].astype(jnp.float32)
            if s < N_DEV - 2:
                send_r[s + 1, :, :] = acc_r.astype(jnp.bfloat16)
                send_l[s + 1, :, :] = acc_l.astype(jnp.bfloat16)

        out_ref[:, :half] = acc_r / (1.0 + jnp.exp(-jnp.clip(acc_r, -60.0, 60.0)))
        out_ref[:, half:] = acc_l / (1.0 + jnp.exp(-jnp.clip(acc_l, -60.0, 60.0)))

    return pl.pallas_call(
        body,
        out_shape=jax.ShapeDtypeStruct((chunk, n), jnp.float32),
        in_specs=[
            pl.BlockSpec(memory_space=pltpu.VMEM),
            pl.BlockSpec(memory_space=pltpu.VMEM),
        ],
        out_specs=pl.BlockSpec(memory_space=pltpu.VMEM),
        scratch_shapes=[
            pltpu.VMEM((N_DEV - 1, chunk, half), jnp.bfloat16),
            pltpu.VMEM((N_DEV - 1, chunk, half), jnp.bfloat16),
            pltpu.VMEM((N_DEV - 1, chunk, half), jnp.bfloat16),
            pltpu.VMEM((N_DEV - 1, chunk, half), jnp.bfloat16),
            pltpu.SemaphoreType.DMA((N_DEV - 1,)),
            pltpu.SemaphoreType.DMA((N_DEV - 1,)),
            pltpu.SemaphoreType.DMA((N_DEV - 1,)),
            pltpu.SemaphoreType.DMA((N_DEV - 1,)),
        ],
        compiler_params=pltpu.CompilerParams(collective_id=0),
    )(x, w_mat)


# device time: 47557 ns/iter; 2.1791x vs baseline; 1.4201x over previous
import jax
import jax.numpy as jnp
from jax import lax
from jax.experimental import pallas as pl
from jax.experimental.pallas import tpu as pltpu

N_DEV = 8

MASK_X, MASK_Y, MASK_Z = 1, 3, 4
ORDERS = (
    (MASK_X, MASK_Y, MASK_Z),
    (MASK_Y, MASK_Z, MASK_X),
    (MASK_Z, MASK_X, MASK_Y),
)
COL_OFF = (0, 768, 1408)
COL_W = (768, 640, 640)
N_SLICE = 3


def kernel(x, w_mat):
    m, k = x.shape
    _, n = w_mat.shape
    chunk = m // N_DEV

    def body(x_ref, w_ref, out_ref, p_ref,
             r0_0, r0_1, r0_2, a4_0, a4_1, a4_2,
             r1_0, r1_1, r1_2, a2_0, a2_1, a2_2,
             r2_0, r2_1, r2_2,
             send_sems, recv_sems):
        r0 = (r0_0, r0_1, r0_2)
        a4 = (a4_0, a4_1, a4_2)
        r1 = (r1_0, r1_1, r1_2)
        a2 = (a2_0, a2_1, a2_2)
        r2 = (r2_0, r2_1, r2_2)

        my = lax.axis_index("i")

        barrier_sem = pltpu.get_barrier_semaphore()
        for mask in (MASK_X, MASK_Y, MASK_Z):
            pl.semaphore_signal(
                barrier_sem, inc=1,
                device_id=(my ^ mask,), device_id_type=pl.DeviceIdType.MESH,
            )
        pl.semaphore_wait(barrier_sem, 3)

        p_ref[:, :] = lax.dot_general(
            x_ref[:, :], w_ref[:, :],
            (((1,), (0,)), ((), ())),
            preferred_element_type=jnp.float32,
        ).astype(jnp.bfloat16)

        def rdma(src, dst, sem_idx, target):
            return pltpu.make_async_remote_copy(
                src_ref=src, dst_ref=dst,
                send_sem=send_sems.at[sem_idx],
                recv_sem=recv_sems.at[sem_idx],
                device_id=(target,),
                device_id_type=pl.DeviceIdType.MESH,
            )

        def f32(v):
            return v.astype(jnp.float32)

        all_descs = []

        p0 = {}
        for j in range(N_SLICE):
            a, b, c = ORDERS[j]
            span_bc = (0, b, c, b ^ c)
            partner = my ^ a
            off, w = COL_OFF[j], COL_W[j]
            for i, s in enumerate(span_bc):
                dest = partner ^ s
                d = rdma(
                    p_ref.at[pl.ds(dest * chunk, chunk), pl.ds(off, w)],
                    r0[j].at[i], 7 * j + i, partner,
                )
                d.start()
                p0[(j, i)] = d
                all_descs.append(d)
        for j in range(N_SLICE):
            a, b, c = ORDERS[j]
            span_bc = (0, b, c, b ^ c)
            off, w = COL_OFF[j], COL_W[j]
            for i, s in enumerate(span_bc):
                p0[(j, i)].wait_recv()
                mine = (my ^ s) * chunk
                a4[j][i, :, :] = (
                    f32(p_ref[pl.ds(mine, chunk), pl.ds(off, w)])
                    + f32(r0[j][i, :, :])
                ).astype(jnp.bfloat16)

        p1 = {}
        for j in range(N_SLICE):
            a, b, c = ORDERS[j]
            partner = my ^ b
            for t, src_i in enumerate((1, 3)):
                d = rdma(a4[j].at[src_i], r1[j].at[t], 7 * j + 4 + t, partner)
                d.start()
                p1[(j, t)] = d
                all_descs.append(d)
        for j in range(N_SLICE):
            p1[(j, 0)].wait_recv()
            p1[(j, 1)].wait_recv()
            a2[j][0, :, :] = (f32(a4[j][0, :, :]) + f32(r1[j][0, :, :])).astype(jnp.bfloat16)
            a2[j][1, :, :] = (f32(a4[j][2, :, :]) + f32(r1[j][1, :, :])).astype(jnp.bfloat16)

        p2 = {}
        for j in range(N_SLICE):
            a, b, c = ORDERS[j]
            partner = my ^ c
            d = rdma(a2[j].at[1], r2[j], 7 * j + 6, partner)
            d.start()
            p2[j] = d
            all_descs.append(d)
        for j in range(N_SLICE):
            off, w = COL_OFF[j], COL_W[j]
            p2[j].wait_recv()
            acc = f32(a2[j][0, :, :]) + f32(r2[j][:, :])
            out_ref[:, off:off + w] = acc / (1.0 + jnp.exp(-jnp.clip(acc, -60.0, 60.0)))

        for d in all_descs:
            d.wait_send()

    return pl.pallas_call(
        body,
        out_shape=jax.ShapeDtypeStruct((chunk, n), jnp.float32),
        in_specs=[
            pl.BlockSpec(memory_space=pltpu.VMEM),
            pl.BlockSpec(memory_space=pltpu.VMEM),
        ],
        out_specs=pl.BlockSpec(memory_space=pltpu.VMEM),
        scratch_shapes=[
            pltpu.VMEM((m, n), jnp.bfloat16),
            pltpu.VMEM((4, chunk, COL_W[0]), jnp.bfloat16),
            pltpu.VMEM((4, chunk, COL_W[1]), jnp.bfloat16),
            pltpu.VMEM((4, chunk, COL_W[2]), jnp.bfloat16),
            pltpu.VMEM((4, chunk, COL_W[0]), jnp.bfloat16),
            pltpu.VMEM((4, chunk, COL_W[1]), jnp.bfloat16),
            pltpu.VMEM((4, chunk, COL_W[2]), jnp.bfloat16),
            pltpu.VMEM((2, chunk, COL_W[0]), jnp.bfloat16),
            pltpu.VMEM((2, chunk, COL_W[1]), jnp.bfloat16),
            pltpu.VMEM((2, chunk, COL_W[2]), jnp.bfloat16),
            pltpu.VMEM((2, chunk, COL_W[0]), jnp.bfloat16),
            pltpu.VMEM((2, chunk, COL_W[1]), jnp.bfloat16),
            pltpu.VMEM((2, chunk, COL_W[2]), jnp.bfloat16),
            pltpu.VMEM((chunk, COL_W[0]), jnp.bfloat16),
            pltpu.VMEM((chunk, COL_W[1]), jnp.bfloat16),
            pltpu.VMEM((chunk, COL_W[2]), jnp.bfloat16),
            pltpu.SemaphoreType.DMA((7 * N_SLICE,)),
            pltpu.SemaphoreType.DMA((7 * N_SLICE,)),
        ],
        compiler_params=pltpu.CompilerParams(collective_id=0),
    )(x, w_mat)


# device time: 45113 ns/iter; 2.2971x vs baseline; 1.0542x over previous
import jax
import jax.numpy as jnp
from jax import lax
from jax.experimental import pallas as pl
from jax.experimental.pallas import tpu as pltpu

N_DEV = 8

MASK_X, MASK_Y, MASK_Z = 1, 3, 4
ORDERS = (
    (MASK_X, MASK_Y, MASK_Z),
    (MASK_Y, MASK_Z, MASK_X),
    (MASK_Z, MASK_X, MASK_Y),
)
COL_OFF = (0, 768, 1408)
COL_W = (768, 640, 640)
N_SLICE = 3
SLICE_ORDER = (1, 2, 0)


def kernel(x, w_mat):
    m, k = x.shape
    _, n = w_mat.shape
    chunk = m // N_DEV

    def body(x_ref, w_ref, out_ref,
             s0_0, s0_1, s0_2, r0_0, r0_1, r0_2,
             a4_0, a4_1, a4_2, r1_0, r1_1, r1_2,
             a2_0, a2_1, a2_2, r2_0, r2_1, r2_2,
             send_sems, recv_sems):
        s0 = (s0_0, s0_1, s0_2)
        r0 = (r0_0, r0_1, r0_2)
        a4 = (a4_0, a4_1, a4_2)
        r1 = (r1_0, r1_1, r1_2)
        a2 = (a2_0, a2_1, a2_2)
        r2 = (r2_0, r2_1, r2_2)

        my = lax.axis_index("i")

        barrier_sem = pltpu.get_barrier_semaphore()
        for mask in (MASK_X, MASK_Y, MASK_Z):
            pl.semaphore_signal(
                barrier_sem, inc=1,
                device_id=(my ^ mask,), device_id_type=pl.DeviceIdType.MESH,
            )
        pl.semaphore_wait(barrier_sem, 3)

        def ptile(dest, j):
            xs = x_ref[pl.ds(dest * chunk, chunk), :]
            return lax.dot_general(
                xs, w_ref[:, COL_OFF[j]:COL_OFF[j] + COL_W[j]],
                (((1,), (0,)), ((), ())),
                preferred_element_type=jnp.float32,
            )

        def rdma(src, dst, sem_idx, target):
            return pltpu.make_async_remote_copy(
                src_ref=src, dst_ref=dst,
                send_sem=send_sems.at[sem_idx],
                recv_sem=recv_sems.at[sem_idx],
                device_id=(target,),
                device_id_type=pl.DeviceIdType.MESH,
            )

        def f32(v):
            return v.astype(jnp.float32)

        spans = []
        for j in range(N_SLICE):
            a, b, c = ORDERS[j]
            spans.append((0, b, c, b ^ c))

        all_descs = []

        p0 = {}
        for j in range(N_SLICE):
            a = ORDERS[j][0]
            partner = my ^ a
            for i, s in enumerate(spans[j]):
                s0[j][i, :, :] = ptile(partner ^ s, j).astype(jnp.bfloat16)
            for i in range(4):
                d = rdma(s0[j].at[i], r0[j].at[i], 7 * j + i, partner)
                d.start()
                p0[(j, i)] = d
                all_descs.append(d)

        kept = {}
        for j in range(N_SLICE):
            for i, s in enumerate(spans[j]):
                kept[(j, i)] = ptile(my ^ s, j)

        p1 = {}
        for j in SLICE_ORDER:
            b = ORDERS[j][1]
            partner = my ^ b
            for i in (1, 3):
                p0[(j, i)].wait_recv()
                a4[j][i, :, :] = (kept[(j, i)] + f32(r0[j][i, :, :])).astype(jnp.bfloat16)
            for t, src_i in enumerate((1, 3)):
                d = rdma(a4[j].at[src_i], r1[j].at[t], 7 * j + 4 + t, partner)
                d.start()
                p1[(j, t)] = d
                all_descs.append(d)
            for i in (0, 2):
                p0[(j, i)].wait_recv()
                a4[j][i, :, :] = (kept[(j, i)] + f32(r0[j][i, :, :])).astype(jnp.bfloat16)

        p2 = {}
        for j in SLICE_ORDER:
            c = ORDERS[j][2]
            partner = my ^ c
            p1[(j, 1)].wait_recv()
            a2[j][1, :, :] = (f32(a4[j][2, :, :]) + f32(r1[j][1, :, :])).astype(jnp.bfloat16)
            d = rdma(a2[j].at[1], r2[j], 7 * j + 6, partner)
            d.start()
            p2[j] = d
            all_descs.append(d)
            p1[(j, 0)].wait_recv()
            a2[j][0, :, :] = (f32(a4[j][0, :, :]) + f32(r1[j][0, :, :])).astype(jnp.bfloat16)

        for j in SLICE_ORDER:
            off, w = COL_OFF[j], COL_W[j]
            p2[j].wait_recv()
            acc = f32(a2[j][0, :, :]) + f32(r2[j][:, :])
            out_ref[:, off:off + w] = acc / (1.0 + jnp.exp(-jnp.clip(acc, -60.0, 60.0)))

        for d in all_descs:
            d.wait_send()

    return pl.pallas_call(
        body,
        out_shape=jax.ShapeDtypeStruct((chunk, n), jnp.float32),
        in_specs=[
            pl.BlockSpec(memory_space=pltpu.VMEM),
            pl.BlockSpec(memory_space=pltpu.VMEM),
        ],
        out_specs=pl.BlockSpec(memory_space=pltpu.VMEM),
        scratch_shapes=[
            pltpu.VMEM((4, chunk, COL_W[0]), jnp.bfloat16),
            pltpu.VMEM((4, chunk, COL_W[1]), jnp.bfloat16),
            pltpu.VMEM((4, chunk, COL_W[2]), jnp.bfloat16),
            pltpu.VMEM((4, chunk, COL_W[0]), jnp.bfloat16),
            pltpu.VMEM((4, chunk, COL_W[1]), jnp.bfloat16),
            pltpu.VMEM((4, chunk, COL_W[2]), jnp.bfloat16),
            pltpu.VMEM((4, chunk, COL_W[0]), jnp.bfloat16),
            pltpu.VMEM((4, chunk, COL_W[1]), jnp.bfloat16),
            pltpu.VMEM((4, chunk, COL_W[2]), jnp.bfloat16),
            pltpu.VMEM((2, chunk, COL_W[0]), jnp.bfloat16),
            pltpu.VMEM((2, chunk, COL_W[1]), jnp.bfloat16),
            pltpu.VMEM((2, chunk, COL_W[2]), jnp.bfloat16),
            pltpu.VMEM((2, chunk, COL_W[0]), jnp.bfloat16),
            pltpu.VMEM((2, chunk, COL_W[1]), jnp.bfloat16),
            pltpu.VMEM((2, chunk, COL_W[2]), jnp.bfloat16),
            pltpu.VMEM((chunk, COL_W[0]), jnp.bfloat16),
            pltpu.VMEM((chunk, COL_W[1]), jnp.bfloat16),
            pltpu.VMEM((chunk, COL_W[2]), jnp.bfloat16),
            pltpu.SemaphoreType.DMA((7 * N_SLICE,)),
            pltpu.SemaphoreType.DMA((7 * N_SLICE,)),
        ],
        compiler_params=pltpu.CompilerParams(collective_id=0),
    )(x, w_mat)


# device time: 42985 ns/iter; 2.4109x vs baseline; 1.0495x over previous
import jax
import jax.numpy as jnp
from jax import lax
from jax.experimental import pallas as pl
from jax.experimental.pallas import tpu as pltpu

N_DEV = 8

MASK_X, MASK_Y, MASK_Z = 1, 3, 4
ORDERS = (
    (MASK_X, MASK_Y, MASK_Z),
    (MASK_Y, MASK_Z, MASK_X),
    (MASK_Z, MASK_X, MASK_Y),
)
COL_OFF = (0, 768, 1408)
COL_W = (768, 640, 640)
N_SLICE = 3
N_STRIP = 2
SLICE_ORDER = (1, 2, 0)


def kernel(x, w_mat):
    m, k = x.shape
    _, n = w_mat.shape
    chunk = m // N_DEV

    def body(x_ref, w_ref, out_ref,
             s0_0, s0_1, s0_2, r0_0, r0_1, r0_2,
             a4_0, a4_1, a4_2, r1_0, r1_1, r1_2,
             a2_0, a2_1, a2_2, r2_0, r2_1, r2_2,
             send_sems, recv_sems):
        s0 = (s0_0, s0_1, s0_2)
        r0 = (r0_0, r0_1, r0_2)
        a4 = (a4_0, a4_1, a4_2)
        r1 = (r1_0, r1_1, r1_2)
        a2 = (a2_0, a2_1, a2_2)
        r2 = (r2_0, r2_1, r2_2)

        my = lax.axis_index("i")

        barrier_sem = pltpu.get_barrier_semaphore()
        for mask in (MASK_X, MASK_Y, MASK_Z):
            pl.semaphore_signal(
                barrier_sem, inc=1,
                device_id=(my ^ mask,), device_id_type=pl.DeviceIdType.MESH,
            )
        pl.semaphore_wait(barrier_sem, 3)

        def ptile(dest, j):
            xs = x_ref[pl.ds(dest * chunk, chunk), :]
            return lax.dot_general(
                xs, w_ref[:, COL_OFF[j]:COL_OFF[j] + COL_W[j]],
                (((1,), (0,)), ((), ())),
                preferred_element_type=jnp.float32,
            )

        def sem_idx(j, h, o):
            return (j * N_STRIP + h) * 7 + o

        def rdma(src, dst, idx, target):
            return pltpu.make_async_remote_copy(
                src_ref=src, dst_ref=dst,
                send_sem=send_sems.at[idx],
                recv_sem=recv_sems.at[idx],
                device_id=(target,),
                device_id_type=pl.DeviceIdType.MESH,
            )

        def f32(v):
            return v.astype(jnp.float32)

        spans = []
        for j in range(N_SLICE):
            a, b, c = ORDERS[j]
            spans.append((0, b, c, b ^ c))
        hws = tuple(w // N_STRIP for w in COL_W)

        all_descs = []

        p0 = {}
        for j in range(N_SLICE):
            a = ORDERS[j][0]
            partner = my ^ a
            hw = hws[j]
            for i, s in enumerate(spans[j]):
                tile = ptile(partner ^ s, j).astype(jnp.bfloat16)
                for h in range(N_STRIP):
                    s0[j][h, i, :, :] = tile[:, h * hw:(h + 1) * hw]
            for h in range(N_STRIP):
                for i in range(4):
                    d = rdma(s0[j].at[h, i], r0[j].at[h, i],
                             sem_idx(j, h, i), partner)
                    d.start()
                    p0[(j, h, i)] = d
                    all_descs.append(d)

        kept = {}
        for j in range(N_SLICE):
            for i, s in enumerate(spans[j]):
                kept[(j, i)] = ptile(my ^ s, j)

        p1 = {}
        for h in range(N_STRIP):
            for j in SLICE_ORDER:
                b = ORDERS[j][1]
                partner = my ^ b
                hw = hws[j]
                cs = pl.ds(h * hw, hw)
                for i in (1, 3):
                    p0[(j, h, i)].wait_recv()
                    a4[j][h, i, :, :] = (
                        kept[(j, i)][:, h * hw:(h + 1) * hw]
                        + f32(r0[j][h, i, :, :])
                    ).astype(jnp.bfloat16)
                for t, src_i in enumerate((1, 3)):
                    d = rdma(a4[j].at[h, src_i], r1[j].at[h, t],
                             sem_idx(j, h, 4 + t), partner)
                    d.start()
                    p1[(j, h, t)] = d
                    all_descs.append(d)
                for i in (0, 2):
                    p0[(j, h, i)].wait_recv()
                    a4[j][h, i, :, :] = (
                        kept[(j, i)][:, h * hw:(h + 1) * hw]
                        + f32(r0[j][h, i, :, :])
                    ).astype(jnp.bfloat16)

        p2 = {}
        for h in range(N_STRIP):
            for j in SLICE_ORDER:
                c = ORDERS[j][2]
                partner = my ^ c
                p1[(j, h, 1)].wait_recv()
                a2[j][h, 1, :, :] = (
                    f32(a4[j][h, 2, :, :]) + f32(r1[j][h, 1, :, :])
                ).astype(jnp.bfloat16)
                d = rdma(a2[j].at[h, 1], r2[j].at[h],
                         sem_idx(j, h, 6), partner)
                d.start()
                p2[(j, h)] = d
                all_descs.append(d)
                p1[(j, h, 0)].wait_recv()
                a2[j][h, 0, :, :] = (
                    f32(a4[j][h, 0, :, :]) + f32(r1[j][h, 0, :, :])
                ).astype(jnp.bfloat16)

        for h in range(N_STRIP):
            for j in SLICE_ORDER:
                hw = hws[j]
                off = COL_OFF[j] + h * hw
                p2[(j, h)].wait_recv()
                acc = f32(a2[j][h, 0, :, :]) + f32(r2[j][h, :, :])
                out_ref[:, off:off + hw] = (
                    acc / (1.0 + jnp.exp(-jnp.clip(acc, -60.0, 60.0)))
                )

        for d in all_descs:
            d.wait_send()

    return pl.pallas_call(
        body,
        out_shape=jax.ShapeDtypeStruct((chunk, n), jnp.float32),
        in_specs=[
            pl.BlockSpec(memory_space=pltpu.VMEM),
            pl.BlockSpec(memory_space=pltpu.VMEM),
        ],
        out_specs=pl.BlockSpec(memory_space=pltpu.VMEM),
        scratch_shapes=[
            pltpu.VMEM((N_STRIP, 4, chunk, COL_W[0] // N_STRIP), jnp.bfloat16),
            pltpu.VMEM((N_STRIP, 4, chunk, COL_W[1] // N_STRIP), jnp.bfloat16),
            pltpu.VMEM((N_STRIP, 4, chunk, COL_W[2] // N_STRIP), jnp.bfloat16),
            pltpu.VMEM((N_STRIP, 4, chunk, COL_W[0] // N_STRIP), jnp.bfloat16),
            pltpu.VMEM((N_STRIP, 4, chunk, COL_W[1] // N_STRIP), jnp.bfloat16),
            pltpu.VMEM((N_STRIP, 4, chunk, COL_W[2] // N_STRIP), jnp.bfloat16),
            pltpu.VMEM((N_STRIP, 4, chunk, COL_W[0] // N_STRIP), jnp.bfloat16),
            pltpu.VMEM((N_STRIP, 4, chunk, COL_W[1] // N_STRIP), jnp.bfloat16),
            pltpu.VMEM((N_STRIP, 4, chunk, COL_W[2] // N_STRIP), jnp.bfloat16),
            pltpu.VMEM((N_STRIP, 2, chunk, COL_W[0] // N_STRIP), jnp.bfloat16),
            pltpu.VMEM((N_STRIP, 2, chunk, COL_W[1] // N_STRIP), jnp.bfloat16),
            pltpu.VMEM((N_STRIP, 2, chunk, COL_W[2] // N_STRIP), jnp.bfloat16),
            pltpu.VMEM((N_STRIP, 2, chunk, COL_W[0] // N_STRIP), jnp.bfloat16),
            pltpu.VMEM((N_STRIP, 2, chunk, COL_W[1] // N_STRIP), jnp.bfloat16),
            pltpu.VMEM((N_STRIP, 2, chunk, COL_W[2] // N_STRIP), jnp.bfloat16),
            pltpu.VMEM((N_STRIP, chunk, COL_W[0] // N_STRIP), jnp.bfloat16),
            pltpu.VMEM((N_STRIP, chunk, COL_W[1] // N_STRIP), jnp.bfloat16),
            pltpu.VMEM((N_STRIP, chunk, COL_W[2] // N_STRIP), jnp.bfloat16),
            pltpu.SemaphoreType.DMA((7 * N_SLICE * N_STRIP,)),
            pltpu.SemaphoreType.DMA((7 * N_SLICE * N_STRIP,)),
        ],
        compiler_params=pltpu.CompilerParams(collective_id=0),
    )(x, w_mat)


# device time: 41877 ns/iter; 2.4747x vs baseline; 1.0265x over previous
import jax
import jax.numpy as jnp
from jax import lax
from jax.experimental import pallas as pl
from jax.experimental.pallas import tpu as pltpu

N_DEV = 8

MASK_X, MASK_Y, MASK_Z = 1, 3, 4
ORDERS = (
    (MASK_X, MASK_Y, MASK_Z),
    (MASK_Y, MASK_Z, MASK_X),
    (MASK_Z, MASK_X, MASK_Y),
)
COL_OFF = (0, 768, 1408)
COL_W = (768, 640, 640)
N_SLICE = 3
N_STRIP = 2
SLICE_ORDER = (1, 2, 0)


def kernel(x, w_mat):
    m, k = x.shape
    _, n = w_mat.shape
    chunk = m // N_DEV

    def body(x_ref, w_ref, out_ref,
             s0_0, s0_1, s0_2, r0_0, r0_1, r0_2,
             a4_0, a4_1, a4_2, r1_0, r1_1, r1_2,
             a2_0, a2_1, a2_2, r2_0, r2_1, r2_2,
             send_sems, recv_sems):
        s0 = (s0_0, s0_1, s0_2)
        r0 = (r0_0, r0_1, r0_2)
        a4 = (a4_0, a4_1, a4_2)
        r1 = (r1_0, r1_1, r1_2)
        a2 = (a2_0, a2_1, a2_2)
        r2 = (r2_0, r2_1, r2_2)

        my = lax.axis_index("i")

        barrier_sem = pltpu.get_barrier_semaphore()
        for mask in (MASK_X, MASK_Y, MASK_Z):
            pl.semaphore_signal(
                barrier_sem, inc=1,
                device_id=(my ^ mask,), device_id_type=pl.DeviceIdType.MESH,
            )
        pl.semaphore_wait(barrier_sem, 3)

        def ptile(dest, j):
            xs = x_ref[pl.ds(dest * chunk, chunk), :]
            return lax.dot_general(
                xs, w_ref[:, COL_OFF[j]:COL_OFF[j] + COL_W[j]],
                (((1,), (0,)), ((), ())),
                preferred_element_type=jnp.float32,
            )

        def sem_idx(j, h, o):
            return (j * N_STRIP + h) * 7 + o

        def rdma(src, dst, idx, target):
            return pltpu.make_async_remote_copy(
                src_ref=src, dst_ref=dst,
                send_sem=send_sems.at[idx],
                recv_sem=recv_sems.at[idx],
                device_id=(target,),
                device_id_type=pl.DeviceIdType.MESH,
            )

        def f32(v):
            return v.astype(jnp.float32)

        spans = []
        for j in range(N_SLICE):
            a, b, c = ORDERS[j]
            spans.append((0, b, c, b ^ c))
        hws = tuple(w // N_STRIP for w in COL_W)

        all_descs = []

        p0 = {}
        for i in range(4):
            for j in range(N_SLICE):
                a = ORDERS[j][0]
                partner = my ^ a
                hw = hws[j]
                s = spans[j][i]
                tile = ptile(partner ^ s, j).astype(jnp.bfloat16)
                for h in range(N_STRIP):
                    s0[j][h, i, :, :] = tile[:, h * hw:(h + 1) * hw]
                d = rdma(s0[j].at[0, i], r0[j].at[0, i],
                         sem_idx(j, 0, i), partner)
                d.start()
                p0[(j, 0, i)] = d
                all_descs.append(d)
        for i in range(4):
            for j in range(N_SLICE):
                partner = my ^ ORDERS[j][0]
                d = rdma(s0[j].at[1, i], r0[j].at[1, i],
                         sem_idx(j, 1, i), partner)
                d.start()
                p0[(j, 1, i)] = d
                all_descs.append(d)

        kept = {}
        for idx_group in ((1, 3), (0, 2)):
            for j in SLICE_ORDER:
                for i in idx_group:
                    kept[(j, i)] = ptile(my ^ spans[j][i], j)

        p1 = {}
        for h in range(N_STRIP):
            for j in SLICE_ORDER:
                b = ORDERS[j][1]
                partner = my ^ b
                hw = hws[j]
                cs = pl.ds(h * hw, hw)
                for i in (1, 3):
                    p0[(j, h, i)].wait_recv()
                    a4[j][h, i, :, :] = (
                        kept[(j, i)][:, h * hw:(h + 1) * hw]
                        + f32(r0[j][h, i, :, :])
                    ).astype(jnp.bfloat16)
                for t, src_i in enumerate((1, 3)):
                    d = rdma(a4[j].at[h, src_i], r1[j].at[h, t],
                             sem_idx(j, h, 4 + t), partner)
                    d.start()
                    p1[(j, h, t)] = d
                    all_descs.append(d)
                for i in (0, 2):
                    p0[(j, h, i)].wait_recv()
                    a4[j][h, i, :, :] = (
                        kept[(j, i)][:, h * hw:(h + 1) * hw]
                        + f32(r0[j][h, i, :, :])
                    ).astype(jnp.bfloat16)

        p2 = {}
        for h in range(N_STRIP):
            for j in SLICE_ORDER:
                c = ORDERS[j][2]
                partner = my ^ c
                p1[(j, h, 1)].wait_recv()
                a2[j][h, 1, :, :] = (
                    f32(a4[j][h, 2, :, :]) + f32(r1[j][h, 1, :, :])
                ).astype(jnp.bfloat16)
                d = rdma(a2[j].at[h, 1], r2[j].at[h],
                         sem_idx(j, h, 6), partner)
                d.start()
                p2[(j, h)] = d
                all_descs.append(d)
                p1[(j, h, 0)].wait_recv()
                a2[j][h, 0, :, :] = (
                    f32(a4[j][h, 0, :, :]) + f32(r1[j][h, 0, :, :])
                ).astype(jnp.bfloat16)

        for h in range(N_STRIP):
            for j in SLICE_ORDER:
                hw = hws[j]
                off = COL_OFF[j] + h * hw
                p2[(j, h)].wait_recv()
                acc = f32(a2[j][h, 0, :, :]) + f32(r2[j][h, :, :])
                out_ref[:, off:off + hw] = (
                    acc / (1.0 + jnp.exp(-jnp.clip(acc, -60.0, 60.0)))
                )

        for d in all_descs:
            d.wait_send()

    return pl.pallas_call(
        body,
        out_shape=jax.ShapeDtypeStruct((chunk, n), jnp.float32),
        in_specs=[
            pl.BlockSpec(memory_space=pltpu.VMEM),
            pl.BlockSpec(memory_space=pltpu.VMEM),
        ],
        out_specs=pl.BlockSpec(memory_space=pltpu.VMEM),
        scratch_shapes=[
            pltpu.VMEM((N_STRIP, 4, chunk, COL_W[0] // N_STRIP), jnp.bfloat16),
            pltpu.VMEM((N_STRIP, 4, chunk, COL_W[1] // N_STRIP), jnp.bfloat16),
            pltpu.VMEM((N_STRIP, 4, chunk, COL_W[2] // N_STRIP), jnp.bfloat16),
            pltpu.VMEM((N_STRIP, 4, chunk, COL_W[0] // N_STRIP), jnp.bfloat16),
            pltpu.VMEM((N_STRIP, 4, chunk, COL_W[1] // N_STRIP), jnp.bfloat16),
            pltpu.VMEM((N_STRIP, 4, chunk, COL_W[2] // N_STRIP), jnp.bfloat16),
            pltpu.VMEM((N_STRIP, 4, chunk, COL_W[0] // N_STRIP), jnp.bfloat16),
            pltpu.VMEM((N_STRIP, 4, chunk, COL_W[1] // N_STRIP), jnp.bfloat16),
            pltpu.VMEM((N_STRIP, 4, chunk, COL_W[2] // N_STRIP), jnp.bfloat16),
            pltpu.VMEM((N_STRIP, 2, chunk, COL_W[0] // N_STRIP), jnp.bfloat16),
            pltpu.VMEM((N_STRIP, 2, chunk, COL_W[1] // N_STRIP), jnp.bfloat16),
            pltpu.VMEM((N_STRIP, 2, chunk, COL_W[2] // N_STRIP), jnp.bfloat16),
            pltpu.VMEM((N_STRIP, 2, chunk, COL_W[0] // N_STRIP), jnp.bfloat16),
            pltpu.VMEM((N_STRIP, 2, chunk, COL_W[1] // N_STRIP), jnp.bfloat16),
            pltpu.VMEM((N_STRIP, 2, chunk, COL_W[2] // N_STRIP), jnp.bfloat16),
            pltpu.VMEM((N_STRIP, chunk, COL_W[0] // N_STRIP), jnp.bfloat16),
            pltpu.VMEM((N_STRIP, chunk, COL_W[1] // N_STRIP), jnp.bfloat16),
            pltpu.VMEM((N_STRIP, chunk, COL_W[2] // N_STRIP), jnp.bfloat16),
            pltpu.SemaphoreType.DMA((7 * N_SLICE * N_STRIP,)),
            pltpu.SemaphoreType.DMA((7 * N_SLICE * N_STRIP,)),
        ],
        compiler_params=pltpu.CompilerParams(collective_id=0),
    )(x, w_mat)


# device time: 39305 ns/iter; 2.6366x vs baseline; 1.0654x over previous
import jax
import jax.numpy as jnp
from jax import lax
from jax.experimental import pallas as pl
from jax.experimental.pallas import tpu as pltpu

N_DEV = 8

MASK_X, MASK_Y, MASK_Z = 1, 3, 4
ORDERS = (
    (MASK_X, MASK_Y, MASK_Z),
    (MASK_Y, MASK_Z, MASK_X),
    (MASK_Z, MASK_X, MASK_Y),
)
COL_OFF = (0, 768, 1408)
COL_W = (768, 640, 640)
STRIPS = ((384, 384), (256, 384), (256, 384))
MAXW = 384
N_SLICE = 3
N_STRIP = 2
SLICE_ORDER = (1, 2, 0)


def kernel(x, w_mat):
    m, k = x.shape
    _, n = w_mat.shape
    chunk = m // N_DEV

    def body(x_ref, w_ref, out_ref,
             s0_0, s0_1, s0_2, r0_0, r0_1, r0_2,
             r1_0, r1_1, r1_2, r2_0, r2_1, r2_2,
             send_sems, recv_sems):
        s0 = (s0_0, s0_1, s0_2)
        r0 = (r0_0, r0_1, r0_2)
        r1 = (r1_0, r1_1, r1_2)
        r2 = (r2_0, r2_1, r2_2)

        my = lax.axis_index("i")

        barrier_sem = pltpu.get_barrier_semaphore()
        for mask in (MASK_X, MASK_Y, MASK_Z):
            pl.semaphore_signal(
                barrier_sem, inc=1,
                device_id=(my ^ mask,), device_id_type=pl.DeviceIdType.MESH,
            )
        pl.semaphore_wait(barrier_sem, 3)

        def ptile(dest, j):
            xs = x_ref[pl.ds(dest * chunk, chunk), :]
            return lax.dot_general(
                xs, w_ref[:, COL_OFF[j]:COL_OFF[j] + COL_W[j]],
                (((1,), (0,)), ((), ())),
                preferred_element_type=jnp.float32,
            )

        def sem_idx(j, h, o):
            return (j * N_STRIP + h) * 7 + o

        def rdma(src, dst, idx, target):
            return pltpu.make_async_remote_copy(
                src_ref=src, dst_ref=dst,
                send_sem=send_sems.at[idx],
                recv_sem=recv_sems.at[idx],
                device_id=(target,),
                device_id_type=pl.DeviceIdType.MESH,
            )

        def f32(v):
            return v.astype(jnp.float32)

        spans = []
        for j in range(N_SLICE):
            a, b, c = ORDERS[j]
            spans.append((0, b, c, b ^ c))

        def strip_off(j, h):
            return sum(STRIPS[j][:h])

        all_descs = []

        p0 = {}
        for i in range(4):
            for j in range(N_SLICE):
                partner = my ^ ORDERS[j][0]
                s = spans[j][i]
                tile = ptile(partner ^ s, j).astype(jnp.bfloat16)
                for h in range(N_STRIP):
                    o, w = strip_off(j, h), STRIPS[j][h]
                    s0[j][h, i, :, 0:w] = tile[:, o:o + w]
                w = STRIPS[j][0]
                d = rdma(s0[j].at[0, i, :, pl.ds(0, w)],
                         r0[j].at[0, i, :, pl.ds(0, w)],
                         sem_idx(j, 0, i), partner)
                d.start()
                p0[(j, 0, i)] = d
                all_descs.append(d)
        for h in range(1, N_STRIP):
            for i in range(4):
                for j in range(N_SLICE):
                    partner = my ^ ORDERS[j][0]
                    w = STRIPS[j][h]
                    d = rdma(s0[j].at[h, i, :, pl.ds(0, w)],
                             r0[j].at[h, i, :, pl.ds(0, w)],
                             sem_idx(j, h, i), partner)
                    d.start()
                    p0[(j, h, i)] = d
                    all_descs.append(d)

        kept = {}
        for idx_group in ((1, 3), (0, 2)):
            for j in SLICE_ORDER:
                for i in idx_group:
                    kept[(j, i)] = ptile(my ^ spans[j][i], j).astype(jnp.bfloat16)

        p1 = {}
        for h in range(N_STRIP):
            for j in SLICE_ORDER:
                partner = my ^ ORDERS[j][1]
                o, w = strip_off(j, h), STRIPS[j][h]
                for i in (1, 3):
                    p0[(j, h, i)].wait_recv()
                    r0[j][h, i, :, 0:w] = (
                        f32(kept[(j, i)][:, o:o + w])
                        + f32(r0[j][h, i, :, 0:w])
                    ).astype(jnp.bfloat16)
                for t, src_i in enumerate((1, 3)):
                    d = rdma(r0[j].at[h, src_i, :, pl.ds(0, w)],
                             r1[j].at[h, t, :, pl.ds(0, w)],
                             sem_idx(j, h, 4 + t), partner)
                    d.start()
                    p1[(j, h, t)] = d
                    all_descs.append(d)
                for i in (0, 2):
                    p0[(j, h, i)].wait_recv()
                    r0[j][h, i, :, 0:w] = (
                        f32(kept[(j, i)][:, o:o + w])
                        + f32(r0[j][h, i, :, 0:w])
                    ).astype(jnp.bfloat16)

        p2 = {}
        for h in range(N_STRIP):
            for j in SLICE_ORDER:
                partner = my ^ ORDERS[j][2]
                w = STRIPS[j][h]
                p1[(j, h, 1)].wait_recv()
                r1[j][h, 1, :, 0:w] = (
                    f32(r0[j][h, 2, :, 0:w]) + f32(r1[j][h, 1, :, 0:w])
                ).astype(jnp.bfloat16)
                d = rdma(r1[j].at[h, 1, :, pl.ds(0, w)],
                         r2[j].at[h, :, pl.ds(0, w)],
                         sem_idx(j, h, 6), partner)
                d.start()
                p2[(j, h)] = d
                all_descs.append(d)

        for h in range(N_STRIP):
            for j in SLICE_ORDER:
                o, w = strip_off(j, h), STRIPS[j][h]
                p1[(j, h, 0)].wait_recv()
                p2[(j, h)].wait_recv()
                acc = (
                    f32(r0[j][h, 0, :, 0:w])
                    + f32(r1[j][h, 0, :, 0:w])
                    + f32(r2[j][h, :, 0:w])
                )
                off = COL_OFF[j] + o
                out_ref[:, off:off + w] = (
                    acc / (1.0 + jnp.exp(-jnp.clip(acc, -60.0, 60.0)))
                )

        for d in all_descs:
            d.wait_send()

    return pl.pallas_call(
        body,
        out_shape=jax.ShapeDtypeStruct((chunk, n), jnp.float32),
        in_specs=[
            pl.BlockSpec(memory_space=pltpu.VMEM),
            pl.BlockSpec(memory_space=pltpu.VMEM),
        ],
        out_specs=pl.BlockSpec(memory_space=pltpu.VMEM),
        scratch_shapes=[
            pltpu.VMEM((N_STRIP, 4, chunk, MAXW), jnp.bfloat16),
            pltpu.VMEM((N_STRIP, 4, chunk, MAXW), jnp.bfloat16),
            pltpu.VMEM((N_STRIP, 4, chunk, MAXW), jnp.bfloat16),
            pltpu.VMEM((N_STRIP, 4, chunk, MAXW), jnp.bfloat16),
            pltpu.VMEM((N_STRIP, 4, chunk, MAXW), jnp.bfloat16),
            pltpu.VMEM((N_STRIP, 4, chunk, MAXW), jnp.bfloat16),
            pltpu.VMEM((N_STRIP, 2, chunk, MAXW), jnp.bfloat16),
            pltpu.VMEM((N_STRIP, 2, chunk, MAXW), jnp.bfloat16),
            pltpu.VMEM((N_STRIP, 2, chunk, MAXW), jnp.bfloat16),
            pltpu.VMEM((N_STRIP, chunk, MAXW), jnp.bfloat16),
            pltpu.VMEM((N_STRIP, chunk, MAXW), jnp.bfloat16),
            pltpu.VMEM((N_STRIP, chunk, MAXW), jnp.bfloat16),
            pltpu.SemaphoreType.DMA((7 * N_SLICE * N_STRIP,)),
            pltpu.SemaphoreType.DMA((7 * N_SLICE * N_STRIP,)),
        ],
        compiler_params=pltpu.CompilerParams(collective_id=0),
    )(x, w_mat)


# device time: 39274 ns/iter; 2.6387x vs baseline; 1.0008x over previous
import jax
import jax.numpy as jnp
from jax import lax
from jax.experimental import pallas as pl
from jax.experimental.pallas import tpu as pltpu

N_DEV = 8

MASK_X, MASK_Y, MASK_Z = 1, 3, 4
ORDERS = (
    (MASK_X, MASK_Y, MASK_Z),
    (MASK_Y, MASK_Z, MASK_X),
    (MASK_Z, MASK_X, MASK_Y),
)
COL_OFF = (0, 768, 1408)
COL_W = (768, 640, 640)
STRIPS = ((384, 384), (256, 384), (256, 384))
MAXW = 384
N_SLICE = 3
N_STRIP = 2
SLICE_ORDER = (1, 2, 0)


def kernel(x, w_mat):
    m, k = x.shape
    _, n = w_mat.shape
    chunk = m // N_DEV

    def body(x_ref, w_ref, out_ref,
             s0_0, s0_1, s0_2, r0_0, r0_1, r0_2,
             r1_0, r1_1, r1_2, r2_0, r2_1, r2_2,
             send_sems, recv_sems):
        s0 = (s0_0, s0_1, s0_2)
        r0 = (r0_0, r0_1, r0_2)
        r1 = (r1_0, r1_1, r1_2)
        r2 = (r2_0, r2_1, r2_2)

        my = lax.axis_index("i")

        barrier_sem = pltpu.get_barrier_semaphore()
        for mask in (MASK_X, MASK_Y, MASK_Z):
            pl.semaphore_signal(
                barrier_sem, inc=1,
                device_id=(my ^ mask,), device_id_type=pl.DeviceIdType.MESH,
            )
        pl.semaphore_wait(barrier_sem, 3)

        wb = [
            w_ref[:, COL_OFF[j]:COL_OFF[j] + COL_W[j]].astype(jnp.bfloat16)
            for j in range(N_SLICE)
        ]

        def ptile(dest, j):
            xs = x_ref[pl.ds(dest * chunk, chunk), :].astype(jnp.bfloat16)
            return lax.dot_general(
                xs, wb[j],
                (((1,), (0,)), ((), ())),
                preferred_element_type=jnp.float32,
            )

        def sem_idx(j, h, o):
            return (j * N_STRIP + h) * 7 + o

        def rdma(src, dst, idx, target):
            return pltpu.make_async_remote_copy(
                src_ref=src, dst_ref=dst,
                send_sem=send_sems.at[idx],
                recv_sem=recv_sems.at[idx],
                device_id=(target,),
                device_id_type=pl.DeviceIdType.MESH,
            )

        def f32(v):
            return v.astype(jnp.float32)

        spans = []
        for j in range(N_SLICE):
            a, b, c = ORDERS[j]
            spans.append((0, b, c, b ^ c))

        def strip_off(j, h):
            return sum(STRIPS[j][:h])

        all_descs = []

        p0 = {}
        for i in range(4):
            for j in range(N_SLICE):
                partner = my ^ ORDERS[j][0]
                s = spans[j][i]
                tile = ptile(partner ^ s, j).astype(jnp.bfloat16)
                for h in range(N_STRIP):
                    o, w = strip_off(j, h), STRIPS[j][h]
                    s0[j][h, i, :, 0:w] = tile[:, o:o + w]
                w = STRIPS[j][0]
                d = rdma(s0[j].at[0, i, :, pl.ds(0, w)],
                         r0[j].at[0, i, :, pl.ds(0, w)],
                         sem_idx(j, 0, i), partner)
                d.start()
                p0[(j, 0, i)] = d
                all_descs.append(d)
        for h in range(1, N_STRIP):
            for i in range(4):
                for j in range(N_SLICE):
                    partner = my ^ ORDERS[j][0]
                    w = STRIPS[j][h]
                    d = rdma(s0[j].at[h, i, :, pl.ds(0, w)],
                             r0[j].at[h, i, :, pl.ds(0, w)],
                             sem_idx(j, h, i), partner)
                    d.start()
                    p0[(j, h, i)] = d
                    all_descs.append(d)

        kept = {}
        for idx_group in ((1, 3), (0, 2)):
            for j in SLICE_ORDER:
                for i in idx_group:
                    kept[(j, i)] = ptile(my ^ spans[j][i], j).astype(jnp.bfloat16)

        p1 = {}
        for h in range(N_STRIP):
            for j in SLICE_ORDER:
                partner = my ^ ORDERS[j][1]
                o, w = strip_off(j, h), STRIPS[j][h]
                for i in (1, 3):
                    p0[(j, h, i)].wait_recv()
                    r0[j][h, i, :, 0:w] = (
                        f32(kept[(j, i)][:, o:o + w])
                        + f32(r0[j][h, i, :, 0:w])
                    ).astype(jnp.bfloat16)
                for t, src_i in enumerate((1, 3)):
                    d = rdma(r0[j].at[h, src_i, :, pl.ds(0, w)],
                             r1[j].at[h, t, :, pl.ds(0, w)],
                             sem_idx(j, h, 4 + t), partner)
                    d.start()
                    p1[(j, h, t)] = d
                    all_descs.append(d)
                for i in (0, 2):
                    p0[(j, h, i)].wait_recv()
                    r0[j][h, i, :, 0:w] = (
                        f32(kept[(j, i)][:, o:o + w])
                        + f32(r0[j][h, i, :, 0:w])
                    ).astype(jnp.bfloat16)

        p2 = {}
        for h in range(N_STRIP):
            for j in SLICE_ORDER:
                partner = my ^ ORDERS[j][2]
                w = STRIPS[j][h]
                p1[(j, h, 1)].wait_recv()
                r1[j][h, 1, :, 0:w] = (
                    f32(r0[j][h, 2, :, 0:w]) + f32(r1[j][h, 1, :, 0:w])
                ).astype(jnp.bfloat16)
                d = rdma(r1[j].at[h, 1, :, pl.ds(0, w)],
                         r2[j].at[h, :, pl.ds(0, w)],
                         sem_idx(j, h, 6), partner)
                d.start()
                p2[(j, h)] = d
                all_descs.append(d)

        for h in range(N_STRIP):
            for j in SLICE_ORDER:
                o, w = strip_off(j, h), STRIPS[j][h]
                p1[(j, h, 0)].wait_recv()
                p2[(j, h)].wait_recv()
                acc = (
                    f32(r0[j][h, 0, :, 0:w])
                    + f32(r1[j][h, 0, :, 0:w])
                    + f32(r2[j][h, :, 0:w])
                )
                off = COL_OFF[j] + o
                out_ref[:, off:off + w] = (
                    acc / (1.0 + jnp.exp(-jnp.clip(acc, -60.0, 60.0)))
                )

        for d in all_descs:
            d.wait_send()

    return pl.pallas_call(
        body,
        out_shape=jax.ShapeDtypeStruct((chunk, n), jnp.float32),
        in_specs=[
            pl.BlockSpec(memory_space=pltpu.VMEM),
            pl.BlockSpec(memory_space=pltpu.VMEM),
        ],
        out_specs=pl.BlockSpec(memory_space=pltpu.VMEM),
        scratch_shapes=[
            pltpu.VMEM((N_STRIP, 4, chunk, MAXW), jnp.bfloat16),
            pltpu.VMEM((N_STRIP, 4, chunk, MAXW), jnp.bfloat16),
            pltpu.VMEM((N_STRIP, 4, chunk, MAXW), jnp.bfloat16),
            pltpu.VMEM((N_STRIP, 4, chunk, MAXW), jnp.bfloat16),
            pltpu.VMEM((N_STRIP, 4, chunk, MAXW), jnp.bfloat16),
            pltpu.VMEM((N_STRIP, 4, chunk, MAXW), jnp.bfloat16),
            pltpu.VMEM((N_STRIP, 2, chunk, MAXW), jnp.bfloat16),
            pltpu.VMEM((N_STRIP, 2, chunk, MAXW), jnp.bfloat16),
            pltpu.VMEM((N_STRIP, 2, chunk, MAXW), jnp.bfloat16),
            pltpu.VMEM((N_STRIP, chunk, MAXW), jnp.bfloat16),
            pltpu.VMEM((N_STRIP, chunk, MAXW), jnp.bfloat16),
            pltpu.VMEM((N_STRIP, chunk, MAXW), jnp.bfloat16),
            pltpu.SemaphoreType.DMA((7 * N_SLICE * N_STRIP,)),
            pltpu.SemaphoreType.DMA((7 * N_SLICE * N_STRIP,)),
        ],
        compiler_params=pltpu.CompilerParams(collective_id=0),
    )(x, w_mat)


# device time: 38910 ns/iter; 2.6634x vs baseline; 1.0094x over previous
import jax
import jax.numpy as jnp
from jax import lax
from jax.experimental import pallas as pl
from jax.experimental.pallas import tpu as pltpu

N_DEV = 8

MASK_X, MASK_Y, MASK_Z = 1, 3, 4
ORDERS = (
    (MASK_X, MASK_Y, MASK_Z),
    (MASK_Y, MASK_Z, MASK_X),
    (MASK_Z, MASK_X, MASK_Y),
)
COL_OFF = (0, 768, 1408)
COL_W = (768, 640, 640)
STRIPS = ((384, 384), (256, 384), (256, 384))
MAXW = 384
N_SLICE = 3
N_STRIP = 2
SLICE_ORDER = (1, 2, 0)


def kernel(x, w_mat):
    m, k = x.shape
    _, n = w_mat.shape
    chunk = m // N_DEV

    def body(x_ref, w_ref, out_ref,
             s0_0, s0_1, s0_2, r0_0, r0_1, r0_2,
             r1_0, r1_1, r1_2, r2_0, r2_1, r2_2,
             send_sems, recv_sems):
        s0 = (s0_0, s0_1, s0_2)
        r0 = (r0_0, r0_1, r0_2)
        r1 = (r1_0, r1_1, r1_2)
        r2 = (r2_0, r2_1, r2_2)

        my = lax.axis_index("i")

        barrier_sem = pltpu.get_barrier_semaphore()
        for mask in (MASK_X, MASK_Y, MASK_Z):
            pl.semaphore_signal(
                barrier_sem, inc=1,
                device_id=(my ^ mask,), device_id_type=pl.DeviceIdType.MESH,
            )

        wb = [
            w_ref[:, COL_OFF[j]:COL_OFF[j] + COL_W[j]].astype(jnp.bfloat16)
            for j in range(N_SLICE)
        ]

        def ptile(dest, j):
            xs = x_ref[pl.ds(dest * chunk, chunk), :].astype(jnp.bfloat16)
            return lax.dot_general(
                xs, wb[j],
                (((1,), (0,)), ((), ())),
                preferred_element_type=jnp.float32,
            )

        def sem_idx(j, h, o):
            return (j * N_STRIP + h) * 7 + o

        def rdma(src, dst, idx, target):
            return pltpu.make_async_remote_copy(
                src_ref=src, dst_ref=dst,
                send_sem=send_sems.at[idx],
                recv_sem=recv_sems.at[idx],
                device_id=(target,),
                device_id_type=pl.DeviceIdType.MESH,
            )

        def f32(v):
            return v.astype(jnp.float32)

        spans = []
        for j in range(N_SLICE):
            a, b, c = ORDERS[j]
            spans.append((0, b, c, b ^ c))

        def strip_off(j, h):
            return sum(STRIPS[j][:h])

        all_descs = []

        p0 = {}
        for i in range(4):
            for j in range(N_SLICE):
                partner = my ^ ORDERS[j][0]
                s = spans[j][i]
                tile = ptile(partner ^ s, j).astype(jnp.bfloat16)
                for h in range(N_STRIP):
                    o, w = strip_off(j, h), STRIPS[j][h]
                    s0[j][h, i, :, 0:w] = tile[:, o:o + w]
                if i == 0 and j == 0:
                    pl.semaphore_wait(barrier_sem, 3)
                w = STRIPS[j][0]
                d = rdma(s0[j].at[0, i, :, pl.ds(0, w)],
                         r0[j].at[0, i, :, pl.ds(0, w)],
                         sem_idx(j, 0, i), partner)
                d.start()
                p0[(j, 0, i)] = d
                all_descs.append(d)
        for h in range(1, N_STRIP):
            for i in range(4):
                for j in range(N_SLICE):
                    partner = my ^ ORDERS[j][0]
                    w = STRIPS[j][h]
                    d = rdma(s0[j].at[h, i, :, pl.ds(0, w)],
                             r0[j].at[h, i, :, pl.ds(0, w)],
                             sem_idx(j, h, i), partner)
                    d.start()
                    p0[(j, h, i)] = d
                    all_descs.append(d)

        kept = {}
        for idx_group in ((1, 3), (0, 2)):
            for j in SLICE_ORDER:
                for i in idx_group:
                    kept[(j, i)] = ptile(my ^ spans[j][i], j).astype(jnp.bfloat16)

        p1 = {}
        for h in range(N_STRIP):
            for j in SLICE_ORDER:
                partner = my ^ ORDERS[j][1]
                o, w = strip_off(j, h), STRIPS[j][h]
                for i in (1, 3):
                    p0[(j, h, i)].wait_recv()
                    r0[j][h, i, :, 0:w] = (
                        kept[(j, i)][:, o:o + w] + r0[j][h, i, :, 0:w]
                    )
                for t, src_i in enumerate((1, 3)):
                    d = rdma(r0[j].at[h, src_i, :, pl.ds(0, w)],
                             r1[j].at[h, t, :, pl.ds(0, w)],
                             sem_idx(j, h, 4 + t), partner)
                    d.start()
                    p1[(j, h, t)] = d
                    all_descs.append(d)
                for i in (0, 2):
                    p0[(j, h, i)].wait_recv()
                    r0[j][h, i, :, 0:w] = (
                        kept[(j, i)][:, o:o + w] + r0[j][h, i, :, 0:w]
                    )

        p2 = {}
        for h in range(N_STRIP):
            for j in SLICE_ORDER:
                partner = my ^ ORDERS[j][2]
                w = STRIPS[j][h]
                p1[(j, h, 1)].wait_recv()
                r1[j][h, 1, :, 0:w] = (
                    r0[j][h, 2, :, 0:w] + r1[j][h, 1, :, 0:w]
                )
                d = rdma(r1[j].at[h, 1, :, pl.ds(0, w)],
                         r2[j].at[h, :, pl.ds(0, w)],
                         sem_idx(j, h, 6), partner)
                d.start()
                p2[(j, h)] = d
                all_descs.append(d)

        for h in range(N_STRIP):
            for j in SLICE_ORDER:
                o, w = strip_off(j, h), STRIPS[j][h]
                p1[(j, h, 0)].wait_recv()
                p2[(j, h)].wait_recv()
                acc = (
                    f32(r0[j][h, 0, :, 0:w])
                    + f32(r1[j][h, 0, :, 0:w])
                    + f32(r2[j][h, :, 0:w])
                )
                off = COL_OFF[j] + o
                out_ref[:, off:off + w] = (
                    acc / (1.0 + jnp.exp(-jnp.clip(acc, -60.0, 60.0)))
                )

        for d in all_descs:
            d.wait_send()

    return pl.pallas_call(
        body,
        out_shape=jax.ShapeDtypeStruct((chunk, n), jnp.float32),
        in_specs=[
            pl.BlockSpec(memory_space=pltpu.VMEM),
            pl.BlockSpec(memory_space=pltpu.VMEM),
        ],
        out_specs=pl.BlockSpec(memory_space=pltpu.VMEM),
        scratch_shapes=[
            pltpu.VMEM((N_STRIP, 4, chunk, MAXW), jnp.bfloat16),
            pltpu.VMEM((N_STRIP, 4, chunk, MAXW), jnp.bfloat16),
            pltpu.VMEM((N_STRIP, 4, chunk, MAXW), jnp.bfloat16),
            pltpu.VMEM((N_STRIP, 4, chunk, MAXW), jnp.bfloat16),
            pltpu.VMEM((N_STRIP, 4, chunk, MAXW), jnp.bfloat16),
            pltpu.VMEM((N_STRIP, 4, chunk, MAXW), jnp.bfloat16),
            pltpu.VMEM((N_STRIP, 2, chunk, MAXW), jnp.bfloat16),
            pltpu.VMEM((N_STRIP, 2, chunk, MAXW), jnp.bfloat16),
            pltpu.VMEM((N_STRIP, 2, chunk, MAXW), jnp.bfloat16),
            pltpu.VMEM((N_STRIP, chunk, MAXW), jnp.bfloat16),
            pltpu.VMEM((N_STRIP, chunk, MAXW), jnp.bfloat16),
            pltpu.VMEM((N_STRIP, chunk, MAXW), jnp.bfloat16),
            pltpu.SemaphoreType.DMA((7 * N_SLICE * N_STRIP,)),
            pltpu.SemaphoreType.DMA((7 * N_SLICE * N_STRIP,)),
        ],
        compiler_params=pltpu.CompilerParams(collective_id=0),
    )(x, w_mat)


# device time: 38607 ns/iter; 2.6843x vs baseline; 1.0078x over previous
import jax
import jax.numpy as jnp
from jax import lax
from jax.experimental import pallas as pl
from jax.experimental.pallas import tpu as pltpu

N_DEV = 8

MASK_X, MASK_Y, MASK_Z = 1, 3, 4
ORDERS = (
    (MASK_X, MASK_Y, MASK_Z),
    (MASK_Y, MASK_Z, MASK_X),
    (MASK_Z, MASK_X, MASK_Y),
)
COL_OFF = (0, 768, 1408)
COL_W = (768, 640, 640)
STRIPS = ((256, 256, 256), (128, 256, 256), (128, 256, 256))
MAXW = 256
N_SLICE = 3
N_STRIP = 3
SLICE_ORDER = (1, 2, 0)


def kernel(x, w_mat):
    m, k = x.shape
    _, n = w_mat.shape
    chunk = m // N_DEV

    def body(x_ref, w_ref, out_ref,
             s0_0, s0_1, s0_2, r0_0, r0_1, r0_2,
             r1_0, r1_1, r1_2, r2_0, r2_1, r2_2,
             send_sems, recv_sems):
        s0 = (s0_0, s0_1, s0_2)
        r0 = (r0_0, r0_1, r0_2)
        r1 = (r1_0, r1_1, r1_2)
        r2 = (r2_0, r2_1, r2_2)

        my = lax.axis_index("i")

        barrier_sem = pltpu.get_barrier_semaphore()
        for mask in (MASK_X, MASK_Y, MASK_Z):
            pl.semaphore_signal(
                barrier_sem, inc=1,
                device_id=(my ^ mask,), device_id_type=pl.DeviceIdType.MESH,
            )

        wb = [
            w_ref[:, COL_OFF[j]:COL_OFF[j] + COL_W[j]].astype(jnp.bfloat16)
            for j in range(N_SLICE)
        ]

        def ptile(dest, j):
            xs = x_ref[pl.ds(dest * chunk, chunk), :].astype(jnp.bfloat16)
            return lax.dot_general(
                xs, wb[j],
                (((1,), (0,)), ((), ())),
                preferred_element_type=jnp.float32,
            )

        def sem_idx(j, h, o):
            return (j * N_STRIP + h) * 7 + o

        def rdma(src, dst, idx, target):
            return pltpu.make_async_remote_copy(
                src_ref=src, dst_ref=dst,
                send_sem=send_sems.at[idx],
                recv_sem=recv_sems.at[idx],
                device_id=(target,),
                device_id_type=pl.DeviceIdType.MESH,
            )

        def f32(v):
            return v.astype(jnp.float32)

        spans = []
        for j in range(N_SLICE):
            a, b, c = ORDERS[j]
            spans.append((0, b, c, b ^ c))

        def strip_off(j, h):
            return sum(STRIPS[j][:h])

        all_descs = []

        p0 = {}
        for i in range(4):
            for j in range(N_SLICE):
                partner = my ^ ORDERS[j][0]
                s = spans[j][i]
                tile = ptile(partner ^ s, j).astype(jnp.bfloat16)
                for h in range(N_STRIP):
                    o, w = strip_off(j, h), STRIPS[j][h]
                    s0[j][h, i, :, 0:w] = tile[:, o:o + w]
                if i == 0 and j == 0:
                    pl.semaphore_wait(barrier_sem, 3)
                w = STRIPS[j][0]
                d = rdma(s0[j].at[0, i, :, pl.ds(0, w)],
                         r0[j].at[0, i, :, pl.ds(0, w)],
                         sem_idx(j, 0, i), partner)
                d.start()
                p0[(j, 0, i)] = d
                all_descs.append(d)
        for h in range(1, N_STRIP):
            for i in range(4):
                for j in range(N_SLICE):
                    partner = my ^ ORDERS[j][0]
                    w = STRIPS[j][h]
                    d = rdma(s0[j].at[h, i, :, pl.ds(0, w)],
                             r0[j].at[h, i, :, pl.ds(0, w)],
                             sem_idx(j, h, i), partner)
                    d.start()
                    p0[(j, h, i)] = d
                    all_descs.append(d)

        kept = {}
        for idx_group in ((1, 3), (0, 2)):
            for j in SLICE_ORDER:
                for i in idx_group:
                    kept[(j, i)] = ptile(my ^ spans[j][i], j).astype(jnp.bfloat16)

        p1 = {}
        for h in range(N_STRIP):
            for j in SLICE_ORDER:
                partner = my ^ ORDERS[j][1]
                o, w = strip_off(j, h), STRIPS[j][h]
                for i in (1, 3):
                    p0[(j, h, i)].wait_recv()
                    r0[j][h, i, :, 0:w] = (
                        kept[(j, i)][:, o:o + w] + r0[j][h, i, :, 0:w]
                    )
                for t, src_i in enumerate((1, 3)):
                    d = rdma(r0[j].at[h, src_i, :, pl.ds(0, w)],
                             r1[j].at[h, t, :, pl.ds(0, w)],
                             sem_idx(j, h, 4 + t), partner)
                    d.start()
                    p1[(j, h, t)] = d
                    all_descs.append(d)
                for i in (0, 2):
                    p0[(j, h, i)].wait_recv()
                    r0[j][h, i, :, 0:w] = (
                        kept[(j, i)][:, o:o + w] + r0[j][h, i, :, 0:w]
                    )

        p2 = {}
        for h in range(N_STRIP):
            for j in SLICE_ORDER:
                partner = my ^ ORDERS[j][2]
                w = STRIPS[j][h]
                p1[(j, h, 1)].wait_recv()
                r1[j][h, 1, :, 0:w] = (
                    r0[j][h, 2, :, 0:w] + r1[j][h, 1, :, 0:w]
                )
                d = rdma(r1[j].at[h, 1, :, pl.ds(0, w)],
                         r2[j].at[h, :, pl.ds(0, w)],
                         sem_idx(j, h, 6), partner)
                d.start()
                p2[(j, h)] = d
                all_descs.append(d)

        for h in range(N_STRIP):
            for j in SLICE_ORDER:
                o, w = strip_off(j, h), STRIPS[j][h]
                p1[(j, h, 0)].wait_recv()
                p2[(j, h)].wait_recv()
                acc = (
                    f32(r0[j][h, 0, :, 0:w])
                    + f32(r1[j][h, 0, :, 0:w])
                    + f32(r2[j][h, :, 0:w])
                )
                off = COL_OFF[j] + o
                out_ref[:, off:off + w] = (
                    acc / (1.0 + jnp.exp(-jnp.clip(acc, -60.0, 60.0)))
                )

        for d in all_descs:
            d.wait_send()

    return pl.pallas_call(
        body,
        out_shape=jax.ShapeDtypeStruct((chunk, n), jnp.float32),
        in_specs=[
            pl.BlockSpec(memory_space=pltpu.VMEM),
            pl.BlockSpec(memory_space=pltpu.VMEM),
        ],
        out_specs=pl.BlockSpec(memory_space=pltpu.VMEM),
        scratch_shapes=[
            pltpu.VMEM((N_STRIP, 4, chunk, MAXW), jnp.bfloat16),
            pltpu.VMEM((N_STRIP, 4, chunk, MAXW), jnp.bfloat16),
            pltpu.VMEM((N_STRIP, 4, chunk, MAXW), jnp.bfloat16),
            pltpu.VMEM((N_STRIP, 4, chunk, MAXW), jnp.bfloat16),
            pltpu.VMEM((N_STRIP, 4, chunk, MAXW), jnp.bfloat16),
            pltpu.VMEM((N_STRIP, 4, chunk, MAXW), jnp.bfloat16),
            pltpu.VMEM((N_STRIP, 2, chunk, MAXW), jnp.bfloat16),
            pltpu.VMEM((N_STRIP, 2, chunk, MAXW), jnp.bfloat16),
            pltpu.VMEM((N_STRIP, 2, chunk, MAXW), jnp.bfloat16),
            pltpu.VMEM((N_STRIP, chunk, MAXW), jnp.bfloat16),
            pltpu.VMEM((N_STRIP, chunk, MAXW), jnp.bfloat16),
            pltpu.VMEM((N_STRIP, chunk, MAXW), jnp.bfloat16),
            pltpu.SemaphoreType.DMA((7 * N_SLICE * N_STRIP,)),
            pltpu.SemaphoreType.DMA((7 * N_SLICE * N_STRIP,)),
        ],
        compiler_params=pltpu.CompilerParams(collective_id=0),
    )(x, w_mat)


# device time: 38595 ns/iter; 2.6851x vs baseline; 1.0003x over previous
import jax
import jax.numpy as jnp
from jax import lax
from jax.experimental import pallas as pl
from jax.experimental.pallas import tpu as pltpu

N_DEV = 8

MASK_X, MASK_Y, MASK_Z = 1, 3, 4
ORDERS = (
    (MASK_X, MASK_Y, MASK_Z),
    (MASK_Y, MASK_Z, MASK_X),
    (MASK_Z, MASK_X, MASK_Y),
)
COL_OFF = (0, 768, 1408)
COL_W = (768, 640, 640)
STRIPS = ((256, 256, 256), (128, 256, 256), (128, 256, 256))
MAXW = 256
N_SLICE = 3
N_STRIP = 3
SLICE_ORDER = (1, 2, 0)


def kernel(x, w_mat):
    m, k = x.shape
    _, n = w_mat.shape
    chunk = m // N_DEV

    def body(x_ref, w_ref, out_ref, xb_ref,
             s0_0, s0_1, s0_2, r0_0, r0_1, r0_2,
             r1_0, r1_1, r1_2, r2_0, r2_1, r2_2,
             send_sems, recv_sems):
        s0 = (s0_0, s0_1, s0_2)
        r0 = (r0_0, r0_1, r0_2)
        r1 = (r1_0, r1_1, r1_2)
        r2 = (r2_0, r2_1, r2_2)

        my = lax.axis_index("i")

        barrier_sem = pltpu.get_barrier_semaphore()
        for mask in (MASK_X, MASK_Y, MASK_Z):
            pl.semaphore_signal(
                barrier_sem, inc=1,
                device_id=(my ^ mask,), device_id_type=pl.DeviceIdType.MESH,
            )

        xb_ref[:, :] = x_ref[:, :].astype(jnp.bfloat16)
        wb = [
            w_ref[:, COL_OFF[j]:COL_OFF[j] + COL_W[j]].astype(jnp.bfloat16)
            for j in range(N_SLICE)
        ]

        def ptile(dest, j):
            xs = xb_ref[pl.ds(dest * chunk, chunk), :]
            return lax.dot_general(
                xs, wb[j],
                (((1,), (0,)), ((), ())),
                preferred_element_type=jnp.float32,
            )

        def sem_idx(j, h, o):
            return (j * N_STRIP + h) * 7 + o

        def rdma(src, dst, idx, target):
            return pltpu.make_async_remote_copy(
                src_ref=src, dst_ref=dst,
                send_sem=send_sems.at[idx],
                recv_sem=recv_sems.at[idx],
                device_id=(target,),
                device_id_type=pl.DeviceIdType.MESH,
            )

        def f32(v):
            return v.astype(jnp.float32)

        spans = []
        for j in range(N_SLICE):
            a, b, c = ORDERS[j]
            spans.append((0, b, c, b ^ c))

        def strip_off(j, h):
            return sum(STRIPS[j][:h])

        all_descs = []

        p0 = {}
        for i in range(4):
            for j in range(N_SLICE):
                partner = my ^ ORDERS[j][0]
                s = spans[j][i]
                tile = ptile(partner ^ s, j).astype(jnp.bfloat16)
                for h in range(N_STRIP):
                    o, w = strip_off(j, h), STRIPS[j][h]
                    s0[j][h, i, :, 0:w] = tile[:, o:o + w]
                if i == 0 and j == 0:
                    pl.semaphore_wait(barrier_sem, 3)
                w = STRIPS[j][0]
                d = rdma(s0[j].at[0, i, :, pl.ds(0, w)],
                         r0[j].at[0, i, :, pl.ds(0, w)],
                         sem_idx(j, 0, i), partner)
                d.start()
                p0[(j, 0, i)] = d
                all_descs.append(d)
        for h in range(1, N_STRIP):
            for i in range(4):
                for j in range(N_SLICE):
                    partner = my ^ ORDERS[j][0]
                    w = STRIPS[j][h]
                    d = rdma(s0[j].at[h, i, :, pl.ds(0, w)],
                             r0[j].at[h, i, :, pl.ds(0, w)],
                             sem_idx(j, h, i), partner)
                    d.start()
                    p0[(j, h, i)] = d
                    all_descs.append(d)

        kept = {}
        for idx_group in ((1, 3), (0, 2)):
            for j in SLICE_ORDER:
                for i in idx_group:
                    kept[(j, i)] = ptile(my ^ spans[j][i], j).astype(jnp.bfloat16)

        p1 = {}
        for h in range(N_STRIP):
            for j in SLICE_ORDER:
                partner = my ^ ORDERS[j][1]
                o, w = strip_off(j, h), STRIPS[j][h]
                for i in (1, 3):
                    p0[(j, h, i)].wait_recv()
                    r0[j][h, i, :, 0:w] = (
                        kept[(j, i)][:, o:o + w] + r0[j][h, i, :, 0:w]
                    )
                for t, src_i in enumerate((1, 3)):
                    d = rdma(r0[j].at[h, src_i, :, pl.ds(0, w)],
                             r1[j].at[h, t, :, pl.ds(0, w)],
                             sem_idx(j, h, 4 + t), partner)
                    d.start()
                    p1[(j, h, t)] = d
                    all_descs.append(d)
                for i in (0, 2):
                    p0[(j, h, i)].wait_recv()
                    r0[j][h, i, :, 0:w] = (
                        kept[(j, i)][:, o:o + w] + r0[j][h, i, :, 0:w]
                    )

        p2 = {}
        for h in range(N_STRIP):
            for j in SLICE_ORDER:
                partner = my ^ ORDERS[j][2]
                w = STRIPS[j][h]
                p1[(j, h, 1)].wait_recv()
                r1[j][h, 1, :, 0:w] = (
                    r0[j][h, 2, :, 0:w] + r1[j][h, 1, :, 0:w]
                )
                d = rdma(r1[j].at[h, 1, :, pl.ds(0, w)],
                         r2[j].at[h, :, pl.ds(0, w)],
                         sem_idx(j, h, 6), partner)
                d.start()
                p2[(j, h)] = d
                all_descs.append(d)

        for h in range(N_STRIP):
            for j in SLICE_ORDER:
                o, w = strip_off(j, h), STRIPS[j][h]
                p1[(j, h, 0)].wait_recv()
                p2[(j, h)].wait_recv()
                acc = (
                    f32(r0[j][h, 0, :, 0:w])
                    + f32(r1[j][h, 0, :, 0:w])
                    + f32(r2[j][h, :, 0:w])
                )
                off = COL_OFF[j] + o
                out_ref[:, off:off + w] = (
                    acc / (1.0 + jnp.exp(-jnp.clip(acc, -60.0, 60.0)))
                )

        for d in all_descs:
            d.wait_send()

    return pl.pallas_call(
        body,
        out_shape=jax.ShapeDtypeStruct((chunk, n), jnp.float32),
        in_specs=[
            pl.BlockSpec(memory_space=pltpu.VMEM),
            pl.BlockSpec(memory_space=pltpu.VMEM),
        ],
        out_specs=pl.BlockSpec(memory_space=pltpu.VMEM),
        scratch_shapes=[
            pltpu.VMEM((m, k), jnp.bfloat16),
            pltpu.VMEM((N_STRIP, 4, chunk, MAXW), jnp.bfloat16),
            pltpu.VMEM((N_STRIP, 4, chunk, MAXW), jnp.bfloat16),
            pltpu.VMEM((N_STRIP, 4, chunk, MAXW), jnp.bfloat16),
            pltpu.VMEM((N_STRIP, 4, chunk, MAXW), jnp.bfloat16),
            pltpu.VMEM((N_STRIP, 4, chunk, MAXW), jnp.bfloat16),
            pltpu.VMEM((N_STRIP, 4, chunk, MAXW), jnp.bfloat16),
            pltpu.VMEM((N_STRIP, 2, chunk, MAXW), jnp.bfloat16),
            pltpu.VMEM((N_STRIP, 2, chunk, MAXW), jnp.bfloat16),
            pltpu.VMEM((N_STRIP, 2, chunk, MAXW), jnp.bfloat16),
            pltpu.VMEM((N_STRIP, chunk, MAXW), jnp.bfloat16),
            pltpu.VMEM((N_STRIP, chunk, MAXW), jnp.bfloat16),
            pltpu.VMEM((N_STRIP, chunk, MAXW), jnp.bfloat16),
            pltpu.SemaphoreType.DMA((7 * N_SLICE * N_STRIP,)),
            pltpu.SemaphoreType.DMA((7 * N_SLICE * N_STRIP,)),
        ],
        compiler_params=pltpu.CompilerParams(collective_id=0),
    )(x, w_mat)
